# Initial kernel scaffold; baseline (speedup 1.0000x reference)
import jax
import jax.numpy as jnp
from jax.experimental import pallas as pl


def kernel(features, edge_index, W0, W1, W2):
    raise NotImplementedError("write your pallas kernel here")


# speedup vs baseline: 2.7319x; 2.7319x over previous
import functools

import jax
import jax.numpy as jnp
from jax import lax
from jax.experimental import pallas as pl
from jax.experimental.pallas import tpu as pltpu
from jax.experimental.pallas import tpu_sc as plsc

N = 10000
NP = 10240
E = 320000
NT = 32
CH = 128
NCH = 80
EP = NT * NCH * CH
SRC_PAD = NP - 1
DST_PAD = NP - 2
RPS = NP // 16

_F32 = jnp.float32


def _mesh():
    return plsc.VectorSubcoreMesh(core_axis_name="c", subcore_axis_name="s")


def _hist_body(idx_hbm, ones_hbm, zeros_hbm, pdeg_hbm,
               idx_v, ones_v, acc_sh):
    c = lax.axis_index("c")
    s = lax.axis_index("s")
    wid = c * 16 + s
    pltpu.sync_copy(zeros_hbm, acc_sh.at[pl.ds(s * RPS, RPS)])
    pltpu.sync_copy(ones_hbm, ones_v)
    pltpu.sync_copy(idx_hbm.at[wid], idx_v)
    plsc.subcore_barrier()

    def step(j, carry):
        pltpu.sync_copy(ones_v, acc_sh.at[idx_v.at[j]], add=True)
        return carry

    lax.fori_loop(0, NCH, step, 0)
    plsc.subcore_barrier()
    rows = pl.ds(s * RPS, RPS)
    pltpu.sync_copy(acc_sh.at[rows], pdeg_hbm.at[c, rows])


_hist = pl.kernel(
    _hist_body,
    out_type=jax.ShapeDtypeStruct((2, NP, 128), _F32),
    mesh=_mesh(),
    scratch_types=[
        pltpu.VMEM((NCH, CH), jnp.int32),
        pltpu.VMEM((CH, 128), _F32),
        pltpu.VMEM_SHARED((NP, 128), _F32),
    ],
)


def _agg_body(m_hbm, src_hbm, dst_hbm, zeros_hbm, out_hbm,
              src_v, dst_v, rows_v, acc_sh, sem):
    c = lax.axis_index("c")
    s = lax.axis_index("s")
    wid = c * 16 + s
    pltpu.sync_copy(zeros_hbm, acc_sh.at[pl.ds(s * RPS, RPS)])
    pltpu.sync_copy(src_hbm.at[wid], src_v)
    pltpu.sync_copy(dst_hbm.at[wid], dst_v)
    plsc.subcore_barrier()

    def step(j, carry):
        pltpu.async_copy(m_hbm.at[src_v.at[j]], rows_v, sem).wait()
        pltpu.sync_copy(rows_v, acc_sh.at[dst_v.at[j]], add=True)
        return carry

    lax.fori_loop(0, NCH, step, 0)
    plsc.subcore_barrier()
    rows = pl.ds(s * RPS, RPS)
    pltpu.sync_copy(acc_sh.at[rows], out_hbm.at[c, rows])


_agg = pl.kernel(
    _agg_body,
    out_type=jax.ShapeDtypeStruct((2, NP, 128), _F32),
    mesh=_mesh(),
    scratch_types=[
        pltpu.VMEM((NCH, CH), jnp.int32),
        pltpu.VMEM((NCH, CH), jnp.int32),
        pltpu.VMEM((CH, 128), _F32),
        pltpu.VMEM_SHARED((NP, 128), _F32),
        pltpu.SemaphoreType.DMA,
    ],
)


def _scales_body(sdeg_ref, ddeg_ref, os_ref, is_ref, c_ref):
    sd = sdeg_ref[0, :, 0:1] + sdeg_ref[1, :, 0:1]
    dd = ddeg_ref[0, :, 0:1] + ddeg_ref[1, :, 0:1]
    os_ = lax.rsqrt(jnp.maximum(sd, 1.0))
    is_ = lax.rsqrt(jnp.maximum(dd, 1.0))
    os_ref[...] = os_
    is_ref[...] = is_
    c_ref[...] = os_ * is_


def _scales(sdeg, ddeg, block=2048):
    return pl.pallas_call(
        _scales_body,
        grid=(NP // block,),
        in_specs=[pl.BlockSpec((2, block, 128), lambda i: (0, i, 0)),
                  pl.BlockSpec((2, block, 128), lambda i: (0, i, 0))],
        out_specs=tuple(pl.BlockSpec((block, 1), lambda i: (i, 0))
                        for _ in range(3)),
        out_shape=(jax.ShapeDtypeStruct((NP, 1), _F32),) * 3,
    )(sdeg, ddeg)


def _mm1_body(x_ref, s_ref, w_ref, o_ref):
    o_ref[...] = jnp.dot(x_ref[...] * s_ref[...], w_ref[...],
                         preferred_element_type=_F32)


def _mm1(x, svec, w, block=1024):
    d_in, d_out = w.shape
    return pl.pallas_call(
        _mm1_body,
        grid=(NP // block,),
        in_specs=[
            pl.BlockSpec((block, d_in), lambda i: (i, 0)),
            pl.BlockSpec((block, 1), lambda i: (i, 0)),
            pl.BlockSpec((d_in, d_out), lambda i: (0, 0)),
        ],
        out_specs=pl.BlockSpec((block, d_out), lambda i: (i, 0)),
        out_shape=jax.ShapeDtypeStruct((NP, d_out), _F32),
    )(x, svec, w)


def _mm2_body(p_ref, c_ref, w_ref, o_ref):
    h = jnp.maximum(p_ref[0] + p_ref[1], 0.0) * c_ref[...]
    o_ref[...] = jnp.dot(h, w_ref[...], preferred_element_type=_F32)


def _mm2(p, cvec, w, block=1024):
    d_in, d_out = w.shape
    return pl.pallas_call(
        _mm2_body,
        grid=(NP // block,),
        in_specs=[
            pl.BlockSpec((2, block, d_in), lambda i: (0, i, 0)),
            pl.BlockSpec((block, 1), lambda i: (i, 0)),
            pl.BlockSpec((d_in, d_out), lambda i: (0, 0)),
        ],
        out_specs=pl.BlockSpec((block, d_out), lambda i: (i, 0)),
        out_shape=jax.ShapeDtypeStruct((NP, d_out), _F32),
    )(p, cvec, w)


def _final_body(p_ref, is_ref, o_ref):
    o_ref[...] = (p_ref[0, :, :64] + p_ref[1, :, :64]) * is_ref[...]


def _final(p, ivec, block=2048):
    return pl.pallas_call(
        _final_body,
        grid=(NP // block,),
        in_specs=[
            pl.BlockSpec((2, block, 128), lambda i: (0, i, 0)),
            pl.BlockSpec((block, 1), lambda i: (i, 0)),
        ],
        out_specs=pl.BlockSpec((block, 64), lambda i: (i, 0)),
        out_shape=jax.ShapeDtypeStruct((NP, 64), _F32),
    )(p, ivec)


def kernel(features, edge_index, W0, W1, W2):
    x = jnp.pad(features, ((0, NP - N), (0, 0)))
    ei = edge_index.astype(jnp.int32)
    pad = EP - E
    src = jnp.pad(ei[0], (0, pad), constant_values=SRC_PAD).reshape(NT, NCH, CH)
    dst = jnp.pad(ei[1], (0, pad), constant_values=DST_PAD).reshape(NT, NCH, CH)

    ones128 = jnp.ones((CH, 128), _F32)
    zeros128 = jnp.zeros((RPS, 128), _F32)
    W2p = jnp.pad(W2, ((0, 0), (0, 128 - W2.shape[1])))

    sdeg = _hist(src, ones128, zeros128)
    ddeg = _hist(dst, ones128, zeros128)
    out_s, in_s, cvec = _scales(sdeg, ddeg)

    m0 = _mm1(x, out_s, W0)
    p0 = _agg(m0, src, dst, zeros128)
    m1 = _mm2(p0, cvec, W1)
    p1 = _agg(m1, src, dst, zeros128)
    m2 = _mm2(p1, cvec, W2p)
    p2 = _agg(m2, src, dst, zeros128)
    out = _final(p2, in_s)
    return out[:N]


# speedup vs baseline: 6.3214x; 2.3140x over previous
import functools

import jax
import jax.numpy as jnp
from jax import lax
from jax.experimental import pallas as pl
from jax.experimental.pallas import tpu as pltpu
from jax.experimental.pallas import tpu_sc as plsc

N = 10000
NP = 10240
E = 320000
NT = 32
CHH = 128
NCHH = 80
CH = 128
NCH = 80
EP = NT * NCH * CH
RPS = NP // 16

_F32 = jnp.float32


def _mesh():
    return plsc.VectorSubcoreMesh(core_axis_name="c", subcore_axis_name="s")


def _hist_body(idx_hbm, ones_hbm, zeros_hbm, pdeg_hbm,
               idx_v, ones_v, acc_sh):
    c = lax.axis_index("c")
    s = lax.axis_index("s")
    wid = c * 16 + s
    pltpu.sync_copy(zeros_hbm, acc_sh.at[pl.ds(s * RPS, RPS)])
    pltpu.sync_copy(ones_hbm, ones_v)
    pltpu.sync_copy(idx_hbm.at[wid], idx_v)
    plsc.subcore_barrier()

    def step(j, carry):
        pltpu.sync_copy(ones_v, acc_sh.at[idx_v.at[j]], add=True)
        return carry

    lax.fori_loop(0, NCHH, step, 0)
    plsc.subcore_barrier()
    rows = pl.ds(s * RPS, RPS)
    pltpu.sync_copy(acc_sh.at[rows], pdeg_hbm.at[c, rows])


_hist = pl.kernel(
    _hist_body,
    out_type=jax.ShapeDtypeStruct((2, NP, 128), _F32),
    mesh=_mesh(),
    scratch_types=[
        pltpu.VMEM((NCHH, CHH), jnp.int32),
        pltpu.VMEM((CHH, 128), _F32),
        pltpu.VMEM_SHARED((NP, 128), _F32),
    ],
)


def _agg_body(m_hbm, src_hbm, dst_hbm, zeros_hbm, out_hbm,
              src_v, dst_v, rows_v, acc_sh, sem):
    c = lax.axis_index("c")
    s = lax.axis_index("s")
    wid = c * 16 + s
    pltpu.sync_copy(zeros_hbm, acc_sh.at[pl.ds(s * RPS, RPS)])
    pltpu.sync_copy(src_hbm.at[wid], src_v)
    pltpu.sync_copy(dst_hbm.at[wid], dst_v)
    plsc.subcore_barrier()

    def step(j, carry):
        pltpu.async_copy(m_hbm.at[src_v.at[j]], rows_v, sem).wait()
        pltpu.sync_copy(rows_v, acc_sh.at[dst_v.at[j]], add=True)
        return carry

    lax.fori_loop(0, NCH, step, 0)
    plsc.subcore_barrier()
    rows = pl.ds(s * RPS, RPS)
    pltpu.sync_copy(acc_sh.at[rows], out_hbm.at[c, rows])


_agg = pl.kernel(
    _agg_body,
    out_type=jax.ShapeDtypeStruct((2, NP, 128), _F32),
    mesh=_mesh(),
    scratch_types=[
        pltpu.VMEM((NCH, CH), jnp.int32),
        pltpu.VMEM((NCH, CH), jnp.int32),
        pltpu.VMEM((CH, 128), _F32),
        pltpu.VMEM_SHARED((NP, 128), _F32),
        pltpu.SemaphoreType.DMA,
    ],
)


def _scales_body(sdeg_ref, ddeg_ref, os_ref, is_ref, c_ref):
    sd = sdeg_ref[0, :, 0:1] + sdeg_ref[1, :, 0:1]
    dd = ddeg_ref[0, :, 0:1] + ddeg_ref[1, :, 0:1]
    os_ = lax.rsqrt(jnp.maximum(sd, 1.0))
    is_ = lax.rsqrt(jnp.maximum(dd, 1.0))
    os_ref[...] = os_
    is_ref[...] = is_
    c_ref[...] = os_ * is_


def _scales(sdeg, ddeg, block=2048):
    return pl.pallas_call(
        _scales_body,
        grid=(NP // block,),
        in_specs=[pl.BlockSpec((2, block, 128), lambda i: (0, i, 0)),
                  pl.BlockSpec((2, block, 128), lambda i: (0, i, 0))],
        out_specs=tuple(pl.BlockSpec((block, 1), lambda i: (i, 0))
                        for _ in range(3)),
        out_shape=(jax.ShapeDtypeStruct((NP, 1), _F32),) * 3,
    )(sdeg, ddeg)


def _mm1_body(x_ref, s_ref, w_ref, o_ref):
    o_ref[...] = jnp.dot(x_ref[...] * s_ref[...], w_ref[...],
                         preferred_element_type=_F32)


def _mm1(x, svec, w, block=1024):
    d_in, d_out = w.shape
    return pl.pallas_call(
        _mm1_body,
        grid=(NP // block,),
        in_specs=[
            pl.BlockSpec((block, d_in), lambda i: (i, 0)),
            pl.BlockSpec((block, 1), lambda i: (i, 0)),
            pl.BlockSpec((d_in, d_out), lambda i: (0, 0)),
        ],
        out_specs=pl.BlockSpec((block, d_out), lambda i: (i, 0)),
        out_shape=jax.ShapeDtypeStruct((NP, d_out), _F32),
    )(x, svec, w)


def _mm2_body(p_ref, c_ref, w_ref, o_ref):
    h = jnp.maximum(p_ref[0] + p_ref[1], 0.0) * c_ref[...]
    o_ref[...] = jnp.dot(h, w_ref[...], preferred_element_type=_F32)


def _mm2(p, cvec, w, block=1024):
    d_in, d_out = w.shape
    return pl.pallas_call(
        _mm2_body,
        grid=(NP // block,),
        in_specs=[
            pl.BlockSpec((2, block, d_in), lambda i: (0, i, 0)),
            pl.BlockSpec((block, 1), lambda i: (i, 0)),
            pl.BlockSpec((d_in, d_out), lambda i: (0, 0)),
        ],
        out_specs=pl.BlockSpec((block, d_out), lambda i: (i, 0)),
        out_shape=jax.ShapeDtypeStruct((NP, d_out), _F32),
    )(p, cvec, w)


def _final_body(p_ref, is_ref, o_ref):
    o_ref[...] = (p_ref[0, :, :64] + p_ref[1, :, :64]) * is_ref[...]


def _final(p, ivec, block=2048):
    return pl.pallas_call(
        _final_body,
        grid=(NP // block,),
        in_specs=[
            pl.BlockSpec((2, block, 128), lambda i: (0, i, 0)),
            pl.BlockSpec((block, 1), lambda i: (i, 0)),
        ],
        out_specs=pl.BlockSpec((block, 64), lambda i: (i, 0)),
        out_shape=jax.ShapeDtypeStruct((NP, 64), _F32),
    )(p, ivec)


def kernel(features, edge_index, W0, W1, W2):
    x = jnp.pad(features, ((0, NP - N), (0, 0)))
    ei = edge_index.astype(jnp.int32)
    pad = EP - E
    sink = N + (jnp.arange(pad, dtype=jnp.int32) % 128)
    src_flat = jnp.concatenate([ei[0], sink])
    dst_flat = jnp.concatenate([ei[1], sink])
    src = src_flat.reshape(NT, NCH, CH)
    dst = dst_flat.reshape(NT, NCH, CH)
    src_h = src_flat.reshape(NT, NCHH, CHH)
    dst_h = dst_flat.reshape(NT, NCHH, CHH)

    ones128 = jnp.ones((CHH, 128), _F32)
    zeros128 = jnp.zeros((RPS, 128), _F32)
    W2p = jnp.pad(W2, ((0, 0), (0, 128 - W2.shape[1])))

    sdeg = _hist(src_h, ones128, zeros128)
    ddeg = _hist(dst_h, ones128, zeros128)
    out_s, in_s, cvec = _scales(sdeg, ddeg)

    m0 = _mm1(x, out_s, W0)
    p0 = _agg(m0, src, dst, zeros128)
    m1 = _mm2(p0, cvec, W1)
    p1 = _agg(m1, src, dst, zeros128)
    m2 = _mm2(p1, cvec, W2p)
    p2 = _agg(m2, src, dst, zeros128)
    out = _final(p2, in_s)
    return out[:N]


# speedup vs baseline: 7.6578x; 1.2114x over previous
import functools

import jax
import jax.numpy as jnp
from jax import lax
from jax.experimental import pallas as pl
from jax.experimental.pallas import tpu as pltpu
from jax.experimental.pallas import tpu_sc as plsc

N = 10000
NP = 10240
E = 320000
NT = 32
CHH = 128
NCHH = 80
CH = 64
NCH = 160
EP = NT * NCH * CH
RPS = NP // 16

_F32 = jnp.float32


def _mesh():
    return plsc.VectorSubcoreMesh(core_axis_name="c", subcore_axis_name="s")


def _hist_body(idx_hbm, ones_hbm, zeros_hbm, pdeg_hbm,
               idx_v, ones_v, acc_sh):
    c = lax.axis_index("c")
    s = lax.axis_index("s")
    wid = c * 16 + s
    pltpu.sync_copy(zeros_hbm, acc_sh.at[pl.ds(s * RPS, RPS)])
    pltpu.sync_copy(ones_hbm, ones_v)
    pltpu.sync_copy(idx_hbm.at[wid], idx_v)
    plsc.subcore_barrier()

    def step(j, carry):
        pltpu.sync_copy(ones_v, acc_sh.at[idx_v.at[j]], add=True)
        return carry

    lax.fori_loop(0, NCHH, step, 0)
    plsc.subcore_barrier()
    rows = pl.ds(s * RPS, RPS)
    pltpu.sync_copy(acc_sh.at[rows], pdeg_hbm.at[c, rows])


_hist = pl.kernel(
    _hist_body,
    out_type=jax.ShapeDtypeStruct((2, NP, 128), _F32),
    mesh=_mesh(),
    scratch_types=[
        pltpu.VMEM((NCHH, CHH), jnp.int32),
        pltpu.VMEM((CHH, 128), _F32),
        pltpu.VMEM_SHARED((NP, 128), _F32),
    ],
)


def _agg_body(m_hbm, src_hbm, dst_hbm, zeros_hbm, out_hbm,
              src_v, dst_v, rows2, acc_sh, gsem):
    c = lax.axis_index("c")
    s = lax.axis_index("s")
    wid = c * 16 + s
    pltpu.sync_copy(zeros_hbm, acc_sh.at[pl.ds(s * RPS, RPS)])
    pltpu.sync_copy(src_hbm.at[wid], src_v)
    pltpu.sync_copy(dst_hbm.at[wid], dst_v)
    plsc.subcore_barrier()

    def step(jj, carry):
        @pl.when(jj >= 2)
        def _():
            j = jj - 2
            b = lax.rem(j, 2)
            pltpu.make_async_copy(m_hbm.at[src_v.at[j]], rows2.at[b],
                                  gsem).wait()
            pltpu.sync_copy(rows2.at[b], acc_sh.at[dst_v.at[j]], add=True)

        @pl.when(jj < NCH)
        def _():
            pltpu.async_copy(m_hbm.at[src_v.at[jj]],
                             rows2.at[lax.rem(jj, 2)], gsem)

        return carry

    lax.fori_loop(0, NCH + 2, step, 0)
    plsc.subcore_barrier()
    rows = pl.ds(s * RPS, RPS)
    pltpu.sync_copy(acc_sh.at[rows], out_hbm.at[c, rows])


_agg = pl.kernel(
    _agg_body,
    out_type=jax.ShapeDtypeStruct((2, NP, 128), _F32),
    mesh=_mesh(),
    compiler_params=pltpu.CompilerParams(use_tc_tiling_on_sc=False),
    scratch_types=[
        pltpu.VMEM((NCH, CH), jnp.int32),
        pltpu.VMEM((NCH, CH), jnp.int32),
        pltpu.VMEM((2, CH, 128), _F32),
        pltpu.VMEM_SHARED((NP, 128), _F32),
        pltpu.SemaphoreType.DMA,
    ],
)


def _scales_body(sdeg_ref, ddeg_ref, os_ref, is_ref, c_ref):
    sd = sdeg_ref[0, :, 0:1] + sdeg_ref[1, :, 0:1]
    dd = ddeg_ref[0, :, 0:1] + ddeg_ref[1, :, 0:1]
    os_ = lax.rsqrt(jnp.maximum(sd, 1.0))
    is_ = lax.rsqrt(jnp.maximum(dd, 1.0))
    os_ref[...] = os_
    is_ref[...] = is_
    c_ref[...] = os_ * is_


def _scales(sdeg, ddeg, block=2048):
    return pl.pallas_call(
        _scales_body,
        grid=(NP // block,),
        in_specs=[pl.BlockSpec((2, block, 128), lambda i: (0, i, 0)),
                  pl.BlockSpec((2, block, 128), lambda i: (0, i, 0))],
        out_specs=tuple(pl.BlockSpec((block, 1), lambda i: (i, 0))
                        for _ in range(3)),
        out_shape=(jax.ShapeDtypeStruct((NP, 1), _F32),) * 3,
    )(sdeg, ddeg)


def _mm1_body(x_ref, s_ref, w_ref, o_ref):
    o_ref[...] = jnp.dot(x_ref[...] * s_ref[...], w_ref[...],
                         preferred_element_type=_F32)


def _mm1(x, svec, w, block=1024):
    d_in, d_out = w.shape
    return pl.pallas_call(
        _mm1_body,
        grid=(NP // block,),
        in_specs=[
            pl.BlockSpec((block, d_in), lambda i: (i, 0)),
            pl.BlockSpec((block, 1), lambda i: (i, 0)),
            pl.BlockSpec((d_in, d_out), lambda i: (0, 0)),
        ],
        out_specs=pl.BlockSpec((block, d_out), lambda i: (i, 0)),
        out_shape=jax.ShapeDtypeStruct((NP, d_out), _F32),
    )(x, svec, w)


def _mm2_body(p_ref, c_ref, w_ref, o_ref):
    h = jnp.maximum(p_ref[0] + p_ref[1], 0.0) * c_ref[...]
    o_ref[...] = jnp.dot(h, w_ref[...], preferred_element_type=_F32)


def _mm2(p, cvec, w, block=1024):
    d_in, d_out = w.shape
    return pl.pallas_call(
        _mm2_body,
        grid=(NP // block,),
        in_specs=[
            pl.BlockSpec((2, block, d_in), lambda i: (0, i, 0)),
            pl.BlockSpec((block, 1), lambda i: (i, 0)),
            pl.BlockSpec((d_in, d_out), lambda i: (0, 0)),
        ],
        out_specs=pl.BlockSpec((block, d_out), lambda i: (i, 0)),
        out_shape=jax.ShapeDtypeStruct((NP, d_out), _F32),
    )(p, cvec, w)


def _final_body(p_ref, is_ref, o_ref):
    o_ref[...] = (p_ref[0, :, :64] + p_ref[1, :, :64]) * is_ref[...]


def _final(p, ivec, block=2048):
    return pl.pallas_call(
        _final_body,
        grid=(NP // block,),
        in_specs=[
            pl.BlockSpec((2, block, 128), lambda i: (0, i, 0)),
            pl.BlockSpec((block, 1), lambda i: (i, 0)),
        ],
        out_specs=pl.BlockSpec((block, 64), lambda i: (i, 0)),
        out_shape=jax.ShapeDtypeStruct((NP, 64), _F32),
    )(p, ivec)


def kernel(features, edge_index, W0, W1, W2):
    x = jnp.pad(features, ((0, NP - N), (0, 0)))
    ei = edge_index.astype(jnp.int32)
    pad = EP - E
    sink = N + (jnp.arange(pad, dtype=jnp.int32) % 128)
    src_flat = jnp.concatenate([ei[0], sink])
    dst_flat = jnp.concatenate([ei[1], sink])
    src = src_flat.reshape(NT, NCH, CH)
    dst = dst_flat.reshape(NT, NCH, CH)
    src_h = src_flat.reshape(NT, NCHH, CHH)
    dst_h = dst_flat.reshape(NT, NCHH, CHH)

    ones128 = jnp.ones((CHH, 128), _F32)
    zeros128 = jnp.zeros((RPS, 128), _F32)
    W2p = jnp.pad(W2, ((0, 0), (0, 128 - W2.shape[1])))

    sdeg = _hist(src_h, ones128, zeros128)
    ddeg = _hist(dst_h, ones128, zeros128)
    out_s, in_s, cvec = _scales(sdeg, ddeg)

    m0 = _mm1(x, out_s, W0)
    p0 = _agg(m0, src, dst, zeros128)
    m1 = _mm2(p0, cvec, W1)
    p1 = _agg(m1, src, dst, zeros128)
    m2 = _mm2(p1, cvec, W2p)
    p2 = _agg(m2, src, dst, zeros128)
    out = _final(p2, in_s)
    return out[:N]


# speedup vs baseline: 8.4860x; 1.1082x over previous
import functools

import jax
import jax.numpy as jnp
from jax import lax
from jax.experimental import pallas as pl
from jax.experimental.pallas import tpu as pltpu
from jax.experimental.pallas import tpu_sc as plsc

N = 10000
NP = 10240
E = 320000
NT = 32
CHH = 128
NCHH = 80
CH = 64
NCH = 160
EP = NT * NCH * CH
RPS = NP // 16

_F32 = jnp.float32


def _mesh():
    return plsc.VectorSubcoreMesh(core_axis_name="c", subcore_axis_name="s")


def _hist_body(idx_hbm, ones_hbm, zeros_hbm, pdeg_hbm,
               idx_v, ones_v, acc_sh):
    c = lax.axis_index("c")
    s = lax.axis_index("s")
    wid = c * 16 + s
    pltpu.sync_copy(zeros_hbm, acc_sh.at[pl.ds(s * RPS, RPS)])
    pltpu.sync_copy(ones_hbm, ones_v)
    pltpu.sync_copy(idx_hbm.at[wid], idx_v)
    plsc.subcore_barrier()

    def step(j, carry):
        pltpu.sync_copy(ones_v, acc_sh.at[idx_v.at[j]], add=True)
        return carry

    lax.fori_loop(0, NCHH, step, 0)
    plsc.subcore_barrier()
    rows = pl.ds(s * RPS, RPS)
    pltpu.sync_copy(acc_sh.at[rows], pdeg_hbm.at[c, rows])


_hist = pl.kernel(
    _hist_body,
    out_type=jax.ShapeDtypeStruct((2, NP, 128), _F32),
    mesh=_mesh(),
    scratch_types=[
        pltpu.VMEM((NCHH, CHH), jnp.int32),
        pltpu.VMEM((CHH, 128), _F32),
        pltpu.VMEM_SHARED((NP, 128), _F32),
    ],
)


def _agg_body(m_hbm, src_hbm, dst_hbm, zeros_hbm, out_hbm,
              src_v, dst_v, rows4, acc_sh, gsem, ssem):
    c = lax.axis_index("c")
    s = lax.axis_index("s")
    wid = c * 16 + s
    pltpu.sync_copy(zeros_hbm, acc_sh.at[pl.ds(s * RPS, RPS)])
    pltpu.sync_copy(src_hbm.at[wid], src_v)
    pltpu.sync_copy(dst_hbm.at[wid], dst_v)
    plsc.subcore_barrier()

    def step(jj, carry):
        @pl.when((jj >= 2) & (jj < NCH + 2))
        def _():
            j = jj - 2
            b = lax.rem(j, 3)
            pltpu.make_async_copy(m_hbm.at[src_v.at[j]], rows4.at[b],
                                  gsem).wait()
            pltpu.async_copy(rows4.at[b], acc_sh.at[dst_v.at[j]], ssem,
                             add=True)

        @pl.when(jj >= 3)
        def _():
            j = jj - 3
            b = lax.rem(j, 3)
            pltpu.make_async_copy(rows4.at[b], acc_sh.at[dst_v.at[j]],
                                  ssem).wait()

            @pl.when(jj < NCH)
            def _():
                pltpu.async_copy(m_hbm.at[src_v.at[jj]], rows4.at[b], gsem)

        @pl.when(jj < 3)
        def _():
            pltpu.async_copy(m_hbm.at[src_v.at[jj]],
                             rows4.at[lax.rem(jj, 3)], gsem)

        return carry

    lax.fori_loop(0, NCH + 3, step, 0)
    plsc.subcore_barrier()
    rows = pl.ds(s * RPS, RPS)
    pltpu.sync_copy(acc_sh.at[rows], out_hbm.at[c, rows])


_agg = pl.kernel(
    _agg_body,
    out_type=jax.ShapeDtypeStruct((2, NP, 128), _F32),
    mesh=_mesh(),
    compiler_params=pltpu.CompilerParams(use_tc_tiling_on_sc=False),
    scratch_types=[
        pltpu.VMEM((NCH, CH), jnp.int32),
        pltpu.VMEM((NCH, CH), jnp.int32),
        pltpu.VMEM((3, CH, 128), _F32),
        pltpu.VMEM_SHARED((NP, 128), _F32),
        pltpu.SemaphoreType.DMA,
        pltpu.SemaphoreType.DMA,
    ],
)


def _scales_body(sdeg_ref, ddeg_ref, os_ref, is_ref, c_ref):
    sd = sdeg_ref[0, :, 0:1] + sdeg_ref[1, :, 0:1]
    dd = ddeg_ref[0, :, 0:1] + ddeg_ref[1, :, 0:1]
    os_ = lax.rsqrt(jnp.maximum(sd, 1.0))
    is_ = lax.rsqrt(jnp.maximum(dd, 1.0))
    os_ref[...] = os_
    is_ref[...] = is_
    c_ref[...] = os_ * is_


def _scales(sdeg, ddeg, block=2048):
    return pl.pallas_call(
        _scales_body,
        grid=(NP // block,),
        in_specs=[pl.BlockSpec((2, block, 128), lambda i: (0, i, 0)),
                  pl.BlockSpec((2, block, 128), lambda i: (0, i, 0))],
        out_specs=tuple(pl.BlockSpec((block, 1), lambda i: (i, 0))
                        for _ in range(3)),
        out_shape=(jax.ShapeDtypeStruct((NP, 1), _F32),) * 3,
    )(sdeg, ddeg)


def _mm1_body(x_ref, s_ref, w_ref, o_ref):
    o_ref[...] = jnp.dot(x_ref[...] * s_ref[...], w_ref[...],
                         preferred_element_type=_F32)


def _mm1(x, svec, w, block=1024):
    d_in, d_out = w.shape
    return pl.pallas_call(
        _mm1_body,
        grid=(NP // block,),
        in_specs=[
            pl.BlockSpec((block, d_in), lambda i: (i, 0)),
            pl.BlockSpec((block, 1), lambda i: (i, 0)),
            pl.BlockSpec((d_in, d_out), lambda i: (0, 0)),
        ],
        out_specs=pl.BlockSpec((block, d_out), lambda i: (i, 0)),
        out_shape=jax.ShapeDtypeStruct((NP, d_out), _F32),
    )(x, svec, w)


def _mm2_body(p_ref, c_ref, w_ref, o_ref):
    h = jnp.maximum(p_ref[0] + p_ref[1], 0.0) * c_ref[...]
    o_ref[...] = jnp.dot(h, w_ref[...], preferred_element_type=_F32)


def _mm2(p, cvec, w, block=1024):
    d_in, d_out = w.shape
    return pl.pallas_call(
        _mm2_body,
        grid=(NP // block,),
        in_specs=[
            pl.BlockSpec((2, block, d_in), lambda i: (0, i, 0)),
            pl.BlockSpec((block, 1), lambda i: (i, 0)),
            pl.BlockSpec((d_in, d_out), lambda i: (0, 0)),
        ],
        out_specs=pl.BlockSpec((block, d_out), lambda i: (i, 0)),
        out_shape=jax.ShapeDtypeStruct((NP, d_out), _F32),
    )(p, cvec, w)


def _final_body(p_ref, is_ref, o_ref):
    o_ref[...] = (p_ref[0, :, :64] + p_ref[1, :, :64]) * is_ref[...]


def _final(p, ivec, block=2048):
    return pl.pallas_call(
        _final_body,
        grid=(NP // block,),
        in_specs=[
            pl.BlockSpec((2, block, 128), lambda i: (0, i, 0)),
            pl.BlockSpec((block, 1), lambda i: (i, 0)),
        ],
        out_specs=pl.BlockSpec((block, 64), lambda i: (i, 0)),
        out_shape=jax.ShapeDtypeStruct((NP, 64), _F32),
    )(p, ivec)


def kernel(features, edge_index, W0, W1, W2):
    x = jnp.pad(features, ((0, NP - N), (0, 0)))
    ei = edge_index.astype(jnp.int32)
    pad = EP - E
    sink = N + (jnp.arange(pad, dtype=jnp.int32) % 128)
    src_flat = jnp.concatenate([ei[0], sink])
    dst_flat = jnp.concatenate([ei[1], sink])
    src = src_flat.reshape(NT, NCH, CH)
    dst = dst_flat.reshape(NT, NCH, CH)
    src_h = src_flat.reshape(NT, NCHH, CHH)
    dst_h = dst_flat.reshape(NT, NCHH, CHH)

    ones128 = jnp.ones((CHH, 128), _F32)
    zeros128 = jnp.zeros((RPS, 128), _F32)
    W2p = jnp.pad(W2, ((0, 0), (0, 128 - W2.shape[1])))

    sdeg = _hist(src_h, ones128, zeros128)
    ddeg = _hist(dst_h, ones128, zeros128)
    out_s, in_s, cvec = _scales(sdeg, ddeg)

    m0 = _mm1(x, out_s, W0)
    p0 = _agg(m0, src, dst, zeros128)
    m1 = _mm2(p0, cvec, W1)
    p1 = _agg(m1, src, dst, zeros128)
    m2 = _mm2(p1, cvec, W2p)
    p2 = _agg(m2, src, dst, zeros128)
    out = _final(p2, in_s)
    return out[:N]
